# baseline (device time: 160506 ns/iter reference)
import jax
import jax.numpy as jnp
from jax import lax
from jax.experimental import pallas as pl
from jax.experimental.pallas import tpu as pltpu

N_DEV = 8


def kernel(x, W1, W2):
    m, _ = x.shape
    n = W2.shape[1]
    ch = m // N_DEV

    def body(x_ref, w1_ref, w2_ref, out_ref,
             txr, rxr, tx_agw, rx_agw, tx_agc, rx_agc,
             sendr, recvr, send_agw, recv_agw, send_agc, recv_agc,
             credit0, credit1, credit_agw, credit_agc):
        my = lax.axis_index("i")
        left = lax.rem(my - 1 + N_DEV, N_DEV)
        right = lax.rem(my + 1, N_DEV)

        def cidx(off):
            return lax.rem(my + off + 4 * N_DEV, N_DEV)

        def chunk(idx):
            return pl.ds(idx * ch, ch)

        barrier_sem = pltpu.get_barrier_semaphore()
        for nbr in (left, right):
            pl.semaphore_signal(barrier_sem, inc=1, device_id=(nbr,),
                                device_id_type=pl.DeviceIdType.MESH)
        pl.semaphore_wait(barrier_sem, 2)

        def partial_chunk(c):
            hc = jnp.dot(x_ref[chunk(c), :], w1_ref[:, :],
                         preferred_element_type=jnp.float32)
            hc = jnp.maximum(hc, 0.0).astype(jnp.bfloat16)
            return jnp.dot(hc, w2_ref[:, :],
                           preferred_element_type=jnp.float32)

        def send_to(tx, rx, ssem, rsem, tgt):
            return pltpu.make_async_remote_copy(
                src_ref=tx, dst_ref=rx, send_sem=ssem, recv_sem=rsem,
                device_id=(tgt,), device_id_type=pl.DeviceIdType.MESH)

        first = partial_chunk(my)
        out_ref[chunk(my), :] = first
        txr[0, :, :] = first.astype(jnp.bfloat16)

        rs_rdma = []
        for s in range(N_DEV - 1):
            slot = s % 2
            if s >= 2:
                pl.semaphore_wait([credit0, credit1][slot], 1)
            rdma = send_to(txr.at[slot], rxr.at[slot],
                           sendr.at[slot], recvr.at[slot], right)
            rdma.start()
            rs_rdma.append(rdma)
            pc = partial_chunk(cidx(-s - 1))
            rdma.wait_recv()
            if s >= 1:
                rs_rdma[s - 1].wait_send()
            val = pc + rxr[slot, :, :].astype(jnp.float32)
            out_ref[chunk(cidx(-s - 1)), :] = val
            if s < N_DEV - 2:
                txr[(s + 1) % 2, :, :] = val.astype(jnp.bfloat16)
            else:
                tx_agw[:, :] = val.astype(jnp.bfloat16)
            if s < N_DEV - 3:
                pl.semaphore_signal([credit0, credit1][slot], inc=1,
                                    device_id=(left,),
                                    device_id_type=pl.DeviceIdType.MESH)


        for t in range(4):
            if t > 0:
                tx_agw[:, :] = out_ref[chunk(cidx(1 - t)), :].astype(
                    jnp.bfloat16)
                pl.semaphore_wait(credit_agw, 1)
            rdma_cw = send_to(tx_agw, rx_agw, send_agw, recv_agw, right)
            rdma_cw.start()
            if t == 0:
                rs_rdma[N_DEV - 2].wait_send()
            if t < 3:
                tx_agc[:, :] = out_ref[chunk(cidx(1 + t)), :].astype(
                    jnp.bfloat16)
                if t > 0:
                    pl.semaphore_wait(credit_agc, 1)
                rdma_ccw = send_to(tx_agc, rx_agc, send_agc, recv_agc, left)
                rdma_ccw.start()
            rdma_cw.wait()
            out_ref[chunk(cidx(-t)), :] = rx_agw[:, :].astype(jnp.float32)
            if t < 3:
                pl.semaphore_signal(credit_agw, inc=1, device_id=(left,),
                                    device_id_type=pl.DeviceIdType.MESH)
                rdma_ccw.wait()
                out_ref[chunk(cidx(2 + t)), :] = rx_agc[:, :].astype(
                    jnp.float32)
                if t < 2:
                    pl.semaphore_signal(credit_agc, inc=1,
                                        device_id=(right,),
                                        device_id_type=pl.DeviceIdType.MESH)

    return pl.pallas_call(
        body,
        out_shape=jax.ShapeDtypeStruct((m, n), jnp.float32),
        in_specs=[pl.BlockSpec(memory_space=pltpu.VMEM)] * 3,
        out_specs=pl.BlockSpec(memory_space=pltpu.VMEM),
        scratch_shapes=[
            pltpu.VMEM((2, ch, n), jnp.bfloat16),
            pltpu.VMEM((2, ch, n), jnp.bfloat16),
            pltpu.VMEM((ch, n), jnp.bfloat16),
            pltpu.VMEM((ch, n), jnp.bfloat16),
            pltpu.VMEM((ch, n), jnp.bfloat16),
            pltpu.VMEM((ch, n), jnp.bfloat16),
            pltpu.SemaphoreType.DMA((2,)),
            pltpu.SemaphoreType.DMA((2,)),
            pltpu.SemaphoreType.DMA,
            pltpu.SemaphoreType.DMA,
            pltpu.SemaphoreType.DMA,
            pltpu.SemaphoreType.DMA,
            pltpu.SemaphoreType.REGULAR,
            pltpu.SemaphoreType.REGULAR,
            pltpu.SemaphoreType.REGULAR,
            pltpu.SemaphoreType.REGULAR,
        ],
        compiler_params=pltpu.CompilerParams(
            collective_id=0, vmem_limit_bytes=100 * 1024 * 1024),
    )(x.astype(jnp.bfloat16), W1.astype(jnp.bfloat16), W2.astype(jnp.bfloat16))


# device time: 143196 ns/iter; 1.1209x vs baseline; 1.1209x over previous
import jax
import jax.numpy as jnp
from jax import lax
from jax.experimental import pallas as pl
from jax.experimental.pallas import tpu as pltpu

N_DEV = 8
STRIP = 384


def kernel(x, W1, W2):
    m, k = x.shape
    h_dim = W1.shape[1]
    n = W2.shape[1]
    ch = m // N_DEV

    def body(x_ref, w1_ref, w2_ref, out_ref,
             xb, w1b, w2b, stripA, stripB,
             copyA_sems, copyB_sems,
             tx_cw, rx_cw, tx_ccw, rx_ccw,
             send_cw, recv_cw, send_ccw, recv_ccw,
             credit_cw, credit_ccw):
        my = lax.axis_index("i")
        left = lax.rem(my - 1 + N_DEV, N_DEV)
        right = lax.rem(my + 1, N_DEV)

        def cidx(off):
            return lax.rem(my + off + 4 * N_DEV, N_DEV)

        def chunk(idx):
            return pl.ds(idx * ch, ch)

        jobsA = [(w1_ref.at[:, pl.ds(j * STRIP, STRIP)], w1b, j * STRIP)
                 for j in range(h_dim // STRIP)]
        jobsA += [(x_ref.at[:, pl.ds(j * STRIP, STRIP)], xb, j * STRIP)
                  for j in range(k // STRIP)]
        jobsB = [(w2_ref.at[pl.ds(j * STRIP, STRIP), :], w2b, j * STRIP)
                 for j in range(h_dim // STRIP)]

        def strip_copy(jobs, buf, sems, j):
            return pltpu.make_async_copy(jobs[j][0], buf.at[j % 2],
                                         sems.at[j % 2])

        cpsA = [strip_copy(jobsA, stripA, copyA_sems, j) for j in range(2)]
        cpsB = [strip_copy(jobsB, stripB, copyB_sems, j) for j in range(2)]
        for cp in cpsA + cpsB:
            cp.start()

        barrier_sem = pltpu.get_barrier_semaphore()
        for nbr in (left, right):
            pl.semaphore_signal(barrier_sem, inc=1, device_id=(nbr,),
                                device_id_type=pl.DeviceIdType.MESH)
        pl.semaphore_wait(barrier_sem, 2)

        for j in range(len(jobsA)):
            cpsA[j].wait()
            dst, off = jobsA[j][1], jobsA[j][2]
            dst[:, pl.ds(off, STRIP)] = stripA[j % 2].astype(jnp.bfloat16)
            if j + 2 < len(jobsA):
                cp = strip_copy(jobsA, stripA, copyA_sems, j + 2)
                cp.start()
                cpsA.append(cp)
        for j in range(len(jobsB)):
            cpsB[j].wait()
            dst, off = jobsB[j][1], jobsB[j][2]
            dst[pl.ds(off, STRIP), :] = stripB[j % 2].astype(jnp.bfloat16)
            if j + 2 < len(jobsB):
                cp = strip_copy(jobsB, stripB, copyB_sems, j + 2)
                cp.start()
                cpsB.append(cp)

        def compute_chunk(c):
            rows = chunk(c)
            hc = jnp.dot(xb[rows, :], w1b[:, :],
                         preferred_element_type=jnp.float32)
            hc = jnp.maximum(hc, 0.0).astype(jnp.bfloat16)
            out_ref[rows, :] = jnp.dot(hc, w2b[:, :],
                                       preferred_element_type=jnp.float32)

        def send_to(tx, rx, ssem, rsem, tgt):
            return pltpu.make_async_remote_copy(
                src_ref=tx, dst_ref=rx, send_sem=ssem, recv_sem=rsem,
                device_id=(tgt,), device_id_type=pl.DeviceIdType.MESH)

        compute_chunk(my)

        for s in range(N_DEV - 1):
            tx_cw[:, :] = out_ref[chunk(cidx(-s)), :].astype(jnp.bfloat16)
            if s > 0:
                pl.semaphore_wait(credit_cw, 1)
            rdma = send_to(tx_cw, rx_cw, send_cw, recv_cw, right)
            rdma.start()
            compute_chunk(cidx(-s - 1))
            rdma.wait()
            out_ref[chunk(cidx(-s - 1)), :] = (
                out_ref[chunk(cidx(-s - 1)), :]
                + rx_cw[:, :].astype(jnp.float32))
            pl.semaphore_signal(credit_cw, inc=1, device_id=(left,),
                                device_id_type=pl.DeviceIdType.MESH)


        for t in range(4):
            tx_cw[:, :] = out_ref[chunk(cidx(1 - t)), :].astype(jnp.bfloat16)
            pl.semaphore_wait(credit_cw, 1)
            rdma_cw = send_to(tx_cw, rx_cw, send_cw, recv_cw, right)
            rdma_cw.start()
            if t < 3:
                tx_ccw[:, :] = out_ref[chunk(cidx(1 + t)), :].astype(
                    jnp.bfloat16)
                if t > 0:
                    pl.semaphore_wait(credit_ccw, 1)
                rdma_ccw = send_to(tx_ccw, rx_ccw, send_ccw, recv_ccw, left)
                rdma_ccw.start()
            rdma_cw.wait()
            out_ref[chunk(cidx(-t)), :] = rx_cw[:, :].astype(jnp.float32)
            if t < 3:
                pl.semaphore_signal(credit_cw, inc=1, device_id=(left,),
                                    device_id_type=pl.DeviceIdType.MESH)
                rdma_ccw.wait()
                out_ref[chunk(cidx(2 + t)), :] = rx_ccw[:, :].astype(
                    jnp.float32)
                if t < 2:
                    pl.semaphore_signal(credit_ccw, inc=1,
                                        device_id=(right,),
                                        device_id_type=pl.DeviceIdType.MESH)

    return pl.pallas_call(
        body,
        out_shape=jax.ShapeDtypeStruct((m, n), jnp.float32),
        in_specs=[pl.BlockSpec(memory_space=pltpu.HBM)] * 3,
        out_specs=pl.BlockSpec(memory_space=pltpu.VMEM),
        scratch_shapes=[
            pltpu.VMEM((m, k), jnp.bfloat16),
            pltpu.VMEM((k, h_dim), jnp.bfloat16),
            pltpu.VMEM((h_dim, n), jnp.bfloat16),
            pltpu.VMEM((2, m, STRIP), jnp.float32),
            pltpu.VMEM((2, STRIP, n), jnp.float32),
            pltpu.SemaphoreType.DMA((2,)),
            pltpu.SemaphoreType.DMA((2,)),
            pltpu.VMEM((ch, n), jnp.bfloat16),
            pltpu.VMEM((ch, n), jnp.bfloat16),
            pltpu.VMEM((ch, n), jnp.bfloat16),
            pltpu.VMEM((ch, n), jnp.bfloat16),
            pltpu.SemaphoreType.DMA,
            pltpu.SemaphoreType.DMA,
            pltpu.SemaphoreType.DMA,
            pltpu.SemaphoreType.DMA,
            pltpu.SemaphoreType.REGULAR,
            pltpu.SemaphoreType.REGULAR,
        ],
        compiler_params=pltpu.CompilerParams(
            collective_id=0, vmem_limit_bytes=100 * 1024 * 1024),
    )(x, W1, W2)


# device time: 129995 ns/iter; 1.2347x vs baseline; 1.1016x over previous
import jax
import jax.numpy as jnp
from jax import lax
from jax.experimental import pallas as pl
from jax.experimental.pallas import tpu as pltpu

N_DEV = 8
STRIP = 512


def kernel(x, W1, W2):
    m, k = x.shape
    h_dim = W1.shape[1]
    n = W2.shape[1]
    ch = m // N_DEV

    def body(x_ref, w1_ref, w2_ref, out_ref,
             xb, w1b, w2b, stripA, stripB,
             copyA_sems, copyB_sems,
             tx_cw, rx_cw, tx_ccw, rx_ccw,
             send_cw, recv_cw, send_ccw, recv_ccw,
             credit_cw, credit_ccw):
        my = lax.axis_index("i")
        left = lax.rem(my - 1 + N_DEV, N_DEV)
        right = lax.rem(my + 1, N_DEV)

        def cidx(off):
            return lax.rem(my + off + 4 * N_DEV, N_DEV)

        def chunk(idx):
            return pl.ds(idx * ch, ch)

        jobsA = [(w1_ref.at[:, pl.ds(j * STRIP, STRIP)], w1b, j * STRIP)
                 for j in range(h_dim // STRIP)]
        jobsA += [(x_ref.at[:, pl.ds(j * STRIP, STRIP)], xb, j * STRIP)
                  for j in range(k // STRIP)]
        jobsB = [(w2_ref.at[pl.ds(j * STRIP, STRIP), :], w2b, j * STRIP)
                 for j in range(h_dim // STRIP)]

        def strip_copy(jobs, buf, sems, j):
            return pltpu.make_async_copy(jobs[j][0], buf.at[j % 2],
                                         sems.at[j % 2])

        cpsA = [strip_copy(jobsA, stripA, copyA_sems, j) for j in range(2)]
        cpsB = [strip_copy(jobsB, stripB, copyB_sems, j) for j in range(2)]
        for cp in cpsA + cpsB:
            cp.start()

        barrier_sem = pltpu.get_barrier_semaphore()
        for nbr in (left, right):
            pl.semaphore_signal(barrier_sem, inc=1, device_id=(nbr,),
                                device_id_type=pl.DeviceIdType.MESH)
        pl.semaphore_wait(barrier_sem, 2)

        for j in range(len(jobsA)):
            cpsA[j].wait()
            dst, off = jobsA[j][1], jobsA[j][2]
            dst[:, pl.ds(off, STRIP)] = stripA[j % 2].astype(jnp.bfloat16)
            if j + 2 < len(jobsA):
                cp = strip_copy(jobsA, stripA, copyA_sems, j + 2)
                cp.start()
                cpsA.append(cp)
        for j in range(len(jobsB)):
            cpsB[j].wait()
            dst, off = jobsB[j][1], jobsB[j][2]
            dst[pl.ds(off, STRIP), :] = stripB[j % 2].astype(jnp.bfloat16)
            if j + 2 < len(jobsB):
                cp = strip_copy(jobsB, stripB, copyB_sems, j + 2)
                cp.start()
                cpsB.append(cp)

        def compute_chunk(c):
            rows = chunk(c)
            hc = jnp.dot(xb[rows, :], w1b[:, :],
                         preferred_element_type=jnp.float32)
            hc = jnp.maximum(hc, 0.0).astype(jnp.bfloat16)
            out_ref[rows, :] = jnp.dot(hc, w2b[:, :],
                                       preferred_element_type=jnp.float32)

        def send_to(tx, rx, ssem, rsem, tgt):
            return pltpu.make_async_remote_copy(
                src_ref=tx, dst_ref=rx, send_sem=ssem, recv_sem=rsem,
                device_id=(tgt,), device_id_type=pl.DeviceIdType.MESH)

        compute_chunk(my)

        cw = []

        def cw_hop(h, src_chunk):
            if h >= 2:
                cw[h - 2].wait_send()
            tx_cw[h % 2, :, :] = out_ref[chunk(src_chunk), :].astype(
                jnp.bfloat16)
            if h >= 2:
                pl.semaphore_wait(credit_cw, 1)
            r = send_to(tx_cw.at[h % 2], rx_cw.at[h % 2],
                        send_cw.at[h % 2], recv_cw.at[h % 2], right)
            r.start()
            cw.append(r)
            return r

        def cw_credit(h):
            if h <= 8:
                pl.semaphore_signal(credit_cw, inc=1, device_id=(left,),
                                    device_id_type=pl.DeviceIdType.MESH)

        for s in range(N_DEV - 1):
            r = cw_hop(s, cidx(-s))
            compute_chunk(cidx(-s - 1))
            r.wait_recv()
            out_ref[chunk(cidx(-s - 1)), :] = (
                out_ref[chunk(cidx(-s - 1)), :]
                + rx_cw[s % 2, :, :].astype(jnp.float32))
            cw_credit(s)


        for t in range(4):
            h = N_DEV - 1 + t
            r = cw_hop(h, cidx(1 - t))
            if t < 3:
                tx_ccw[:, :] = out_ref[chunk(cidx(1 + t)), :].astype(
                    jnp.bfloat16)
                if t > 0:
                    pl.semaphore_wait(credit_ccw, 1)
                rdma_ccw = send_to(tx_ccw, rx_ccw, send_ccw, recv_ccw, left)
                rdma_ccw.start()
            r.wait_recv()
            out_ref[chunk(cidx(-t)), :] = rx_cw[h % 2, :, :].astype(
                jnp.float32)
            cw_credit(h)
            if t < 3:
                rdma_ccw.wait()
                out_ref[chunk(cidx(2 + t)), :] = rx_ccw[:, :].astype(
                    jnp.float32)
                if t < 2:
                    pl.semaphore_signal(credit_ccw, inc=1,
                                        device_id=(right,),
                                        device_id_type=pl.DeviceIdType.MESH)

        cw[-2].wait_send()
        cw[-1].wait_send()

    return pl.pallas_call(
        body,
        out_shape=jax.ShapeDtypeStruct((m, n), jnp.float32),
        in_specs=[pl.BlockSpec(memory_space=pltpu.HBM)] * 3,
        out_specs=pl.BlockSpec(memory_space=pltpu.VMEM),
        scratch_shapes=[
            pltpu.VMEM((m, k), jnp.bfloat16),
            pltpu.VMEM((k, h_dim), jnp.bfloat16),
            pltpu.VMEM((h_dim, n), jnp.bfloat16),
            pltpu.VMEM((2, m, STRIP), jnp.float32),
            pltpu.VMEM((2, STRIP, n), jnp.float32),
            pltpu.SemaphoreType.DMA((2,)),
            pltpu.SemaphoreType.DMA((2,)),
            pltpu.VMEM((2, ch, n), jnp.bfloat16),
            pltpu.VMEM((2, ch, n), jnp.bfloat16),
            pltpu.VMEM((ch, n), jnp.bfloat16),
            pltpu.VMEM((ch, n), jnp.bfloat16),
            pltpu.SemaphoreType.DMA((2,)),
            pltpu.SemaphoreType.DMA((2,)),
            pltpu.SemaphoreType.DMA,
            pltpu.SemaphoreType.DMA,
            pltpu.SemaphoreType.REGULAR,
            pltpu.SemaphoreType.REGULAR,
        ],
        compiler_params=pltpu.CompilerParams(
            collective_id=0, vmem_limit_bytes=100 * 1024 * 1024),
    )(x, W1, W2)


# device time: 128741 ns/iter; 1.2467x vs baseline; 1.0097x over previous
import jax
import jax.numpy as jnp
from jax import lax
from jax.experimental import pallas as pl
from jax.experimental.pallas import tpu as pltpu

N_DEV = 8
STRIP = 512


def kernel(x, W1, W2):
    m, k = x.shape
    h_dim = W1.shape[1]
    n = W2.shape[1]
    ch = m // N_DEV

    def body(x_ref, w1_ref, w2_ref, out_ref,
             xb, w1b, w2b, hb, stripA, stripB,
             copyA_sems, copyB_sems,
             tx_cw, rx_cw,
             tx_agw, rx_agw, tx_agc, rx_agc,
             send_cw, recv_cw,
             send_agw, recv_agw, send_agc, recv_agc,
             credit_cw, credit_agw, credit_agc):
        my = lax.axis_index("i")
        left = lax.rem(my - 1 + N_DEV, N_DEV)
        right = lax.rem(my + 1, N_DEV)

        def cidx(off):
            return lax.rem(my + off + 4 * N_DEV, N_DEV)

        def chunk(idx):
            return pl.ds(idx * ch, ch)

        jobsA = [(w1_ref.at[:, pl.ds(j * STRIP, STRIP)], w1b, j * STRIP)
                 for j in range(h_dim // STRIP)]
        jobsA += [(x_ref.at[:, pl.ds(j * STRIP, STRIP)], xb, j * STRIP)
                  for j in range(k // STRIP)]
        jobsB = [(w2_ref.at[pl.ds(j * STRIP, STRIP), :], w2b, j * STRIP)
                 for j in range(h_dim // STRIP)]

        def strip_copy(jobs, buf, sems, j):
            return pltpu.make_async_copy(jobs[j][0], buf.at[j % 2],
                                         sems.at[j % 2])

        cpsA = [strip_copy(jobsA, stripA, copyA_sems, j) for j in range(2)]
        cpsB = [strip_copy(jobsB, stripB, copyB_sems, j) for j in range(2)]
        for cp in cpsA + cpsB:
            cp.start()

        barrier_sem = pltpu.get_barrier_semaphore()
        for nbr in (left, right):
            pl.semaphore_signal(barrier_sem, inc=1, device_id=(nbr,),
                                device_id_type=pl.DeviceIdType.MESH)
        pl.semaphore_wait(barrier_sem, 2)

        for j in range(len(jobsA)):
            cpsA[j].wait()
            dst, off = jobsA[j][1], jobsA[j][2]
            dst[:, pl.ds(off, STRIP)] = stripA[j % 2].astype(jnp.bfloat16)
            if j + 2 < len(jobsA):
                cp = strip_copy(jobsA, stripA, copyA_sems, j + 2)
                cp.start()
                cpsA.append(cp)
        hb[:, :] = jnp.maximum(
            jnp.dot(xb[chunk(my), :], w1b[:, :],
                    preferred_element_type=jnp.float32), 0.0
        ).astype(jnp.bfloat16)
        for j in range(len(jobsB)):
            cpsB[j].wait()
            dst, off = jobsB[j][1], jobsB[j][2]
            dst[pl.ds(off, STRIP), :] = stripB[j % 2].astype(jnp.bfloat16)
            if j + 2 < len(jobsB):
                cp = strip_copy(jobsB, stripB, copyB_sems, j + 2)
                cp.start()
                cpsB.append(cp)

        def compute_chunk(c):
            rows = chunk(c)
            hc = jnp.dot(xb[rows, :], w1b[:, :],
                         preferred_element_type=jnp.float32)
            hc = jnp.maximum(hc, 0.0).astype(jnp.bfloat16)
            out_ref[rows, :] = jnp.dot(hc, w2b[:, :],
                                       preferred_element_type=jnp.float32)

        def send_to(tx, rx, ssem, rsem, tgt):
            return pltpu.make_async_remote_copy(
                src_ref=tx, dst_ref=rx, send_sem=ssem, recv_sem=rsem,
                device_id=(tgt,), device_id_type=pl.DeviceIdType.MESH)

        out_ref[chunk(my), :] = jnp.dot(hb[:, :], w2b[:, :],
                                        preferred_element_type=jnp.float32)

        cw = []
        for s in range(N_DEV - 1):
            if s >= 2:
                cw[s - 2].wait_send()
            tx_cw[s % 2, :, :] = out_ref[chunk(cidx(-s)), :].astype(
                jnp.bfloat16)
            if s >= 2:
                pl.semaphore_wait(credit_cw, 1)
            r = send_to(tx_cw.at[s % 2], rx_cw.at[s % 2],
                        send_cw.at[s % 2], recv_cw.at[s % 2], right)
            r.start()
            cw.append(r)
            compute_chunk(cidx(-s - 1))
            r.wait_recv()
            out_ref[chunk(cidx(-s - 1)), :] = (
                out_ref[chunk(cidx(-s - 1)), :]
                + rx_cw[s % 2, :, :].astype(jnp.float32))
            if s <= 4:
                pl.semaphore_signal(credit_cw, inc=1, device_id=(left,),
                                    device_id_type=pl.DeviceIdType.MESH)


        nh = n // 2
        agw = []
        agc = []

        def ag_send(lst, tx, rx, ssems, rsems, credit, tgt,
                    src_chunk, colr):
            j = len(lst)
            if j >= 2:
                lst[j - 2].wait_send()
                pl.semaphore_wait(credit, 1)
            tx[j % 2, :, :] = out_ref[chunk(src_chunk),
                                      pl.ds(colr, nh)].astype(jnp.bfloat16)
            r = send_to(tx.at[j % 2], rx.at[j % 2],
                        ssems.at[j % 2], rsems.at[j % 2], tgt)
            r.start()
            lst.append(r)
            return r

        for t in range(4):
            r0 = ag_send(agw, tx_agw, rx_agw, send_agw, recv_agw,
                         credit_agw, right, cidx(1 - t), 0)
            r1 = ag_send(agw, tx_agw, rx_agw, send_agw, recv_agw,
                         credit_agw, right, cidx(1 - t), nh)
            if t == 0:
                cw[-2].wait_send()
                cw[-1].wait_send()
            if t < 3:
                c0 = ag_send(agc, tx_agc, rx_agc, send_agc, recv_agc,
                             credit_agc, left, cidx(1 + t), 0)
                c1 = ag_send(agc, tx_agc, rx_agc, send_agc, recv_agc,
                             credit_agc, left, cidx(1 + t), nh)
            r0.wait_recv()
            out_ref[chunk(cidx(-t)), pl.ds(0, nh)] = (
                rx_agw[0, :, :].astype(jnp.float32))
            if t <= 2:
                pl.semaphore_signal(credit_agw, inc=1, device_id=(left,),
                                    device_id_type=pl.DeviceIdType.MESH)
            r1.wait_recv()
            out_ref[chunk(cidx(-t)), pl.ds(nh, nh)] = (
                rx_agw[1, :, :].astype(jnp.float32))
            if t <= 2:
                pl.semaphore_signal(credit_agw, inc=1, device_id=(left,),
                                    device_id_type=pl.DeviceIdType.MESH)
            if t < 3:
                c0.wait_recv()
                out_ref[chunk(cidx(2 + t)), pl.ds(0, nh)] = (
                    rx_agc[0, :, :].astype(jnp.float32))
                if t <= 1:
                    pl.semaphore_signal(credit_agc, inc=1,
                                        device_id=(right,),
                                        device_id_type=pl.DeviceIdType.MESH)
                c1.wait_recv()
                out_ref[chunk(cidx(2 + t)), pl.ds(nh, nh)] = (
                    rx_agc[1, :, :].astype(jnp.float32))
                if t <= 1:
                    pl.semaphore_signal(credit_agc, inc=1,
                                        device_id=(right,),
                                        device_id_type=pl.DeviceIdType.MESH)

        agw[-2].wait_send()
        agw[-1].wait_send()
        agc[-2].wait_send()
        agc[-1].wait_send()

    return pl.pallas_call(
        body,
        out_shape=jax.ShapeDtypeStruct((m, n), jnp.float32),
        in_specs=[pl.BlockSpec(memory_space=pltpu.HBM)] * 3,
        out_specs=pl.BlockSpec(memory_space=pltpu.VMEM),
        scratch_shapes=[
            pltpu.VMEM((m, k), jnp.bfloat16),
            pltpu.VMEM((k, h_dim), jnp.bfloat16),
            pltpu.VMEM((h_dim, n), jnp.bfloat16),
            pltpu.VMEM((m // N_DEV, h_dim), jnp.bfloat16),
            pltpu.VMEM((2, m, STRIP), jnp.float32),
            pltpu.VMEM((2, STRIP, n), jnp.float32),
            pltpu.SemaphoreType.DMA((2,)),
            pltpu.SemaphoreType.DMA((2,)),
            pltpu.VMEM((2, ch, n), jnp.bfloat16),
            pltpu.VMEM((2, ch, n), jnp.bfloat16),
            pltpu.VMEM((2, ch, n // 2), jnp.bfloat16),
            pltpu.VMEM((2, ch, n // 2), jnp.bfloat16),
            pltpu.VMEM((2, ch, n // 2), jnp.bfloat16),
            pltpu.VMEM((2, ch, n // 2), jnp.bfloat16),
            pltpu.SemaphoreType.DMA((2,)),
            pltpu.SemaphoreType.DMA((2,)),
            pltpu.SemaphoreType.DMA((2,)),
            pltpu.SemaphoreType.DMA((2,)),
            pltpu.SemaphoreType.DMA((2,)),
            pltpu.SemaphoreType.DMA((2,)),
            pltpu.SemaphoreType.REGULAR,
            pltpu.SemaphoreType.REGULAR,
            pltpu.SemaphoreType.REGULAR,
        ],
        compiler_params=pltpu.CompilerParams(
            collective_id=0, vmem_limit_bytes=100 * 1024 * 1024),
    )(x, W1, W2)


# device time: 113003 ns/iter; 1.4204x vs baseline; 1.1393x over previous
import jax
import jax.numpy as jnp
from jax import lax
from jax.experimental import pallas as pl
from jax.experimental.pallas import tpu as pltpu

N_DEV = 8
STRIP = 512


def kernel(x, W1, W2):
    m, k = x.shape
    h_dim = W1.shape[1]
    n = W2.shape[1]
    ch = m // N_DEV
    nh = n // 2

    def body(x_ref, w1_ref, w2_ref, out_ref,
             xb, w1b, w2b, hb, stripA, stripB,
             copyA_sems, copyB_sems,
             txA, rxA, txB, rxB, txC, rxC, txD, rxD,
             sendA, recvA, sendB, recvB,
             sendC, recvC, sendD, recvD,
             creditA, creditB, creditC, creditD):
        my = lax.axis_index("i")
        left = lax.rem(my - 1 + N_DEV, N_DEV)
        right = lax.rem(my + 1, N_DEV)

        def cidx(off):
            return lax.rem(my + off + 4 * N_DEV, N_DEV)

        def chunk(idx):
            return pl.ds(idx * ch, ch)

        jobsA = [(w1_ref.at[:, pl.ds(j * STRIP, STRIP)], w1b, j * STRIP)
                 for j in range(h_dim // STRIP)]
        jobsA += [(x_ref.at[:, pl.ds(j * STRIP, STRIP)], xb, j * STRIP)
                  for j in range(k // STRIP)]
        jobsB = [(w2_ref.at[pl.ds(j * STRIP, STRIP), :], w2b, j * STRIP)
                 for j in range(h_dim // STRIP)]

        def strip_copy(jobs, buf, sems, j):
            return pltpu.make_async_copy(jobs[j][0], buf.at[j % 2],
                                         sems.at[j % 2])

        cpsA = [strip_copy(jobsA, stripA, copyA_sems, j) for j in range(2)]
        cpsB = [strip_copy(jobsB, stripB, copyB_sems, j) for j in range(2)]
        for cp in cpsA + cpsB:
            cp.start()

        barrier_sem = pltpu.get_barrier_semaphore()
        for nbr in (left, right):
            pl.semaphore_signal(barrier_sem, inc=1, device_id=(nbr,),
                                device_id_type=pl.DeviceIdType.MESH)
        pl.semaphore_wait(barrier_sem, 2)

        for j in range(len(jobsA)):
            cpsA[j].wait()
            dst, off = jobsA[j][1], jobsA[j][2]
            dst[:, pl.ds(off, STRIP)] = stripA[j % 2].astype(jnp.bfloat16)
            if j + 2 < len(jobsA):
                cp = strip_copy(jobsA, stripA, copyA_sems, j + 2)
                cp.start()
                cpsA.append(cp)
        hb[:, :] = jnp.maximum(
            jnp.dot(xb[chunk(my), :], w1b[:, :],
                    preferred_element_type=jnp.float32), 0.0
        ).astype(jnp.bfloat16)
        for j in range(len(jobsB)):
            cpsB[j].wait()
            dst, off = jobsB[j][1], jobsB[j][2]
            dst[pl.ds(off, STRIP), :] = stripB[j % 2].astype(jnp.bfloat16)
            if j + 2 < len(jobsB):
                cp = strip_copy(jobsB, stripB, copyB_sems, j + 2)
                cp.start()
                cpsB.append(cp)

        def compute_chunk(c):
            rows = chunk(c)
            hc = jnp.dot(xb[rows, :], w1b[:, :],
                         preferred_element_type=jnp.float32)
            hc = jnp.maximum(hc, 0.0).astype(jnp.bfloat16)
            out_ref[rows, :] = jnp.dot(hc, w2b[:, :],
                                       preferred_element_type=jnp.float32)

        out_ref[chunk(my), :] = jnp.dot(hb[:, :], w2b[:, :],
                                        preferred_element_type=jnp.float32)

        def make_stream(tx, rx, ssems, rsems, credit, tgt, up, total):
            msgs = []
            ncons = [0]

            def send(src_chunk, col):
                j = len(msgs)
                if j >= 2:
                    msgs[j - 2].wait_send()
                    pl.semaphore_wait(credit, 1)
                tx[j % 2, :, :] = out_ref[chunk(src_chunk),
                                          pl.ds(col, nh)].astype(
                    jnp.bfloat16)
                r = pltpu.make_async_remote_copy(
                    src_ref=tx.at[j % 2], dst_ref=rx.at[j % 2],
                    send_sem=ssems.at[j % 2], recv_sem=rsems.at[j % 2],
                    device_id=(tgt,), device_id_type=pl.DeviceIdType.MESH)
                r.start()
                msgs.append(r)

            def consume(dst_chunk, col, accumulate):
                j = ncons[0]
                msgs[j].wait_recv()
                val = rx[j % 2, :, :].astype(jnp.float32)
                rows, cols = chunk(dst_chunk), pl.ds(col, nh)
                if accumulate:
                    out_ref[rows, cols] = out_ref[rows, cols] + val
                else:
                    out_ref[rows, cols] = val
                if j + 2 < total:
                    pl.semaphore_signal(credit, inc=1, device_id=(up,),
                                        device_id_type=pl.DeviceIdType.MESH)
                ncons[0] += 1

            def drain():
                msgs[-2].wait_send()
                msgs[-1].wait_send()

            return send, consume, drain

        LOW, HIGH = 0, nh
        sendA_, consA, drainA = make_stream(
            txA, rxA, sendA, recvA, creditA, right, left, 11)
        sendB_, consB, drainB = make_stream(
            txB, rxB, sendB, recvB, creditB, left, right, 11)
        sendC_, consC, drainC = make_stream(
            txC, rxC, sendC, recvC, creditC, left, right, 3)
        sendD_, consD, drainD = make_stream(
            txD, rxD, sendD, recvD, creditD, right, left, 3)

        for s in range(N_DEV - 1):
            sendA_(cidx(-s), LOW)
            sendB_(cidx(s), HIGH)
            if s < 3:
                compute_chunk(cidx(s + 1))
                compute_chunk(cidx(-s - 1))
            elif s == 3:
                compute_chunk(cidx(4))
            consA(cidx(-s - 1), LOW, True)
            consB(cidx(s + 1), HIGH, True)


        for t in range(4):
            sendA_(cidx(1 - t), LOW)
            sendB_(cidx(t - 1), HIGH)
            if t < 3:
                sendC_(cidx(1 + t), LOW)
                sendD_(cidx(-1 - t), HIGH)
            consA(cidx(-t), LOW, False)
            consB(cidx(t), HIGH, False)
            if t < 3:
                consC(cidx(2 + t), LOW, False)
                consD(cidx(-2 - t), HIGH, False)

        for drain in (drainA, drainB, drainC, drainD):
            drain()

    return pl.pallas_call(
        body,
        out_shape=jax.ShapeDtypeStruct((m, n), jnp.float32),
        in_specs=[pl.BlockSpec(memory_space=pltpu.HBM)] * 3,
        out_specs=pl.BlockSpec(memory_space=pltpu.VMEM),
        scratch_shapes=[
            pltpu.VMEM((m, k), jnp.bfloat16),
            pltpu.VMEM((k, h_dim), jnp.bfloat16),
            pltpu.VMEM((h_dim, n), jnp.bfloat16),
            pltpu.VMEM((m // N_DEV, h_dim), jnp.bfloat16),
            pltpu.VMEM((2, m, STRIP), jnp.float32),
            pltpu.VMEM((2, STRIP, n), jnp.float32),
            pltpu.SemaphoreType.DMA((2,)),
            pltpu.SemaphoreType.DMA((2,)),
            pltpu.VMEM((2, ch, nh), jnp.bfloat16),
            pltpu.VMEM((2, ch, nh), jnp.bfloat16),
            pltpu.VMEM((2, ch, nh), jnp.bfloat16),
            pltpu.VMEM((2, ch, nh), jnp.bfloat16),
            pltpu.VMEM((2, ch, nh), jnp.bfloat16),
            pltpu.VMEM((2, ch, nh), jnp.bfloat16),
            pltpu.VMEM((2, ch, nh), jnp.bfloat16),
            pltpu.VMEM((2, ch, nh), jnp.bfloat16),
            pltpu.SemaphoreType.DMA((2,)),
            pltpu.SemaphoreType.DMA((2,)),
            pltpu.SemaphoreType.DMA((2,)),
            pltpu.SemaphoreType.DMA((2,)),
            pltpu.SemaphoreType.DMA((2,)),
            pltpu.SemaphoreType.DMA((2,)),
            pltpu.SemaphoreType.DMA((2,)),
            pltpu.SemaphoreType.DMA((2,)),
            pltpu.SemaphoreType.REGULAR,
            pltpu.SemaphoreType.REGULAR,
            pltpu.SemaphoreType.REGULAR,
            pltpu.SemaphoreType.REGULAR,
        ],
        compiler_params=pltpu.CompilerParams(
            collective_id=0, vmem_limit_bytes=100 * 1024 * 1024),
    )(x, W1, W2)


# device time: 109220 ns/iter; 1.4696x vs baseline; 1.0346x over previous
import jax
import jax.numpy as jnp
from jax import lax
from jax.experimental import pallas as pl
from jax.experimental.pallas import tpu as pltpu

N_DEV = 8
STRIP = 512


def kernel(x, W1, W2):
    m, k = x.shape
    h_dim = W1.shape[1]
    n = W2.shape[1]
    ch = m // N_DEV
    nh = n // 2

    def body(x_ref, w1_ref, w2_ref, out_ref,
             xb, w1b, w2b, hb, hcache, stripA, stripB,
             copyA_sems, copyB_sems,
             txA, rxA, txB, rxB, txC, rxC, txD, rxD,
             sendA, recvA, sendB, recvB,
             sendC, recvC, sendD, recvD,
             creditA, creditB, creditC, creditD):
        my = lax.axis_index("i")
        left = lax.rem(my - 1 + N_DEV, N_DEV)
        right = lax.rem(my + 1, N_DEV)

        def cidx(off):
            return lax.rem(my + off + 4 * N_DEV, N_DEV)

        def chunk(idx):
            return pl.ds(idx * ch, ch)

        jobsA = [(w1_ref.at[:, pl.ds(j * STRIP, STRIP)], w1b, j * STRIP)
                 for j in range(h_dim // STRIP)]
        jobsA += [(x_ref.at[:, pl.ds(j * STRIP, STRIP)], xb, j * STRIP)
                  for j in range(k // STRIP)]
        jobsB = [(w2_ref.at[pl.ds(j * STRIP, STRIP), :], w2b, j * STRIP)
                 for j in range(h_dim // STRIP)]

        def strip_copy(jobs, buf, sems, j):
            return pltpu.make_async_copy(jobs[j][0], buf.at[j % 2],
                                         sems.at[j % 2])

        cpsA = [strip_copy(jobsA, stripA, copyA_sems, j) for j in range(2)]
        cpsB = [strip_copy(jobsB, stripB, copyB_sems, j) for j in range(2)]
        for cp in cpsA + cpsB:
            cp.start()

        barrier_sem = pltpu.get_barrier_semaphore()
        for nbr in (left, right):
            pl.semaphore_signal(barrier_sem, inc=1, device_id=(nbr,),
                                device_id_type=pl.DeviceIdType.MESH)
        pl.semaphore_wait(barrier_sem, 2)

        for j in range(len(jobsA)):
            cpsA[j].wait()
            dst, off = jobsA[j][1], jobsA[j][2]
            dst[:, pl.ds(off, STRIP)] = stripA[j % 2].astype(jnp.bfloat16)
            if j + 2 < len(jobsA):
                cp = strip_copy(jobsA, stripA, copyA_sems, j + 2)
                cp.start()
                cpsA.append(cp)
        hb[:, :] = jnp.maximum(
            jnp.dot(xb[chunk(my), :], w1b[:, :],
                    preferred_element_type=jnp.float32), 0.0
        ).astype(jnp.bfloat16)
        for j in range(len(jobsB)):
            cpsB[j].wait()
            dst, off = jobsB[j][1], jobsB[j][2]
            dst[pl.ds(off, STRIP), :] = stripB[j % 2].astype(jnp.bfloat16)
            if j + 2 < len(jobsB):
                cp = strip_copy(jobsB, stripB, copyB_sems, j + 2)
                cp.start()
                cpsB.append(cp)

        def compute_chunk(c):
            rows = chunk(c)
            hc = jnp.dot(xb[rows, :], w1b[:, :],
                         preferred_element_type=jnp.float32)
            hc = jnp.maximum(hc, 0.0).astype(jnp.bfloat16)
            out_ref[rows, :] = jnp.dot(hc, w2b[:, :],
                                       preferred_element_type=jnp.float32)

        out_ref[chunk(my), :] = jnp.dot(hb[:, :], w2b[:, :],
                                        preferred_element_type=jnp.float32)

        def make_stream(tx, rx, ssems, rsems, credit, tgt, up, total):
            msgs = []
            ncons = [0]

            def send(src_chunk, col):
                j = len(msgs)
                if j >= 2:
                    msgs[j - 2].wait_send()
                    pl.semaphore_wait(credit, 1)
                tx[j % 2, :, :] = out_ref[chunk(src_chunk),
                                          pl.ds(col, nh)].astype(
                    jnp.bfloat16)
                r = pltpu.make_async_remote_copy(
                    src_ref=tx.at[j % 2], dst_ref=rx.at[j % 2],
                    send_sem=ssems.at[j % 2], recv_sem=rsems.at[j % 2],
                    device_id=(tgt,), device_id_type=pl.DeviceIdType.MESH)
                r.start()
                msgs.append(r)

            def consume(dst_chunk, col, accumulate):
                j = ncons[0]
                msgs[j].wait_recv()
                val = rx[j % 2, :, :].astype(jnp.float32)
                rows, cols = chunk(dst_chunk), pl.ds(col, nh)
                if accumulate:
                    out_ref[rows, cols] = out_ref[rows, cols] + val
                else:
                    out_ref[rows, cols] = val
                if j + 2 < total:
                    pl.semaphore_signal(credit, inc=1, device_id=(up,),
                                        device_id_type=pl.DeviceIdType.MESH)
                ncons[0] += 1

            def drain():
                msgs[-2].wait_send()
                msgs[-1].wait_send()

            return send, consume, drain

        LOW, HIGH = 0, nh
        sendA_, consA, drainA = make_stream(
            txA, rxA, sendA, recvA, creditA, right, left, 11)
        sendB_, consB, drainB = make_stream(
            txB, rxB, sendB, recvB, creditB, left, right, 11)
        sendC_, consC, drainC = make_stream(
            txC, rxC, sendC, recvC, creditC, left, right, 3)
        sendD_, consD, drainD = make_stream(
            txD, rxD, sendD, recvD, creditD, right, left, 3)

        def h_of(c):
            return jnp.maximum(
                jnp.dot(xb[chunk(c), :], w1b[:, :],
                        preferred_element_type=jnp.float32), 0.0
            ).astype(jnp.bfloat16)

        def half_dot(hval, dst_chunk, col):
            out_ref[chunk(dst_chunk), pl.ds(col, nh)] = jnp.dot(
                hval, w2b[:, pl.ds(col, nh)],
                preferred_element_type=jnp.float32)

        for s in range(N_DEV - 1):
            sendA_(cidx(-s), LOW)
            sendB_(cidx(s), HIGH)
            if s < 3:
                hv = h_of(cidx(s + 1))
                hcache[s, :, :] = hv
                half_dot(hv, cidx(s + 1), HIGH)
                hv = h_of(cidx(-s - 1))
                hcache[3 + s, :, :] = hv
                half_dot(hv, cidx(-s - 1), LOW)
            elif s == 3:
                compute_chunk(cidx(4))
            else:
                q = 6 - s
                half_dot(hcache[q, :, :], cidx(q + 1), LOW)
                half_dot(hcache[3 + q, :, :], cidx(-q - 1), HIGH)
            consA(cidx(-s - 1), LOW, True)
            consB(cidx(s + 1), HIGH, True)


        for t in range(4):
            sendA_(cidx(1 - t), LOW)
            sendB_(cidx(t - 1), HIGH)
            if t < 3:
                sendC_(cidx(1 + t), LOW)
                sendD_(cidx(-1 - t), HIGH)
            consA(cidx(-t), LOW, False)
            consB(cidx(t), HIGH, False)
            if t < 3:
                consC(cidx(2 + t), LOW, False)
                consD(cidx(-2 - t), HIGH, False)

        for drain in (drainA, drainB, drainC, drainD):
            drain()

    return pl.pallas_call(
        body,
        out_shape=jax.ShapeDtypeStruct((m, n), jnp.float32),
        in_specs=[pl.BlockSpec(memory_space=pltpu.HBM)] * 3,
        out_specs=pl.BlockSpec(memory_space=pltpu.VMEM),
        scratch_shapes=[
            pltpu.VMEM((m, k), jnp.bfloat16),
            pltpu.VMEM((k, h_dim), jnp.bfloat16),
            pltpu.VMEM((h_dim, n), jnp.bfloat16),
            pltpu.VMEM((m // N_DEV, h_dim), jnp.bfloat16),
            pltpu.VMEM((6, m // N_DEV, h_dim), jnp.bfloat16),
            pltpu.VMEM((2, m, STRIP), jnp.float32),
            pltpu.VMEM((2, STRIP, n), jnp.float32),
            pltpu.SemaphoreType.DMA((2,)),
            pltpu.SemaphoreType.DMA((2,)),
            pltpu.VMEM((2, ch, nh), jnp.bfloat16),
            pltpu.VMEM((2, ch, nh), jnp.bfloat16),
            pltpu.VMEM((2, ch, nh), jnp.bfloat16),
            pltpu.VMEM((2, ch, nh), jnp.bfloat16),
            pltpu.VMEM((2, ch, nh), jnp.bfloat16),
            pltpu.VMEM((2, ch, nh), jnp.bfloat16),
            pltpu.VMEM((2, ch, nh), jnp.bfloat16),
            pltpu.VMEM((2, ch, nh), jnp.bfloat16),
            pltpu.SemaphoreType.DMA((2,)),
            pltpu.SemaphoreType.DMA((2,)),
            pltpu.SemaphoreType.DMA((2,)),
            pltpu.SemaphoreType.DMA((2,)),
            pltpu.SemaphoreType.DMA((2,)),
            pltpu.SemaphoreType.DMA((2,)),
            pltpu.SemaphoreType.DMA((2,)),
            pltpu.SemaphoreType.DMA((2,)),
            pltpu.SemaphoreType.REGULAR,
            pltpu.SemaphoreType.REGULAR,
            pltpu.SemaphoreType.REGULAR,
            pltpu.SemaphoreType.REGULAR,
        ],
        compiler_params=pltpu.CompilerParams(
            collective_id=0, vmem_limit_bytes=100 * 1024 * 1024),
    )(x, W1, W2)


# device time: 105057 ns/iter; 1.5278x vs baseline; 1.0396x over previous
import jax
import jax.numpy as jnp
from jax import lax
from jax.experimental import pallas as pl
from jax.experimental.pallas import tpu as pltpu

N_DEV = 8
STRIP = 512


def kernel(x, W1, W2):
    m, k = x.shape
    h_dim = W1.shape[1]
    n = W2.shape[1]
    ch = m // N_DEV
    nh = n // 2

    def body(x_ref, w1_ref, w2_ref, out_ref,
             xb, w1b, w2b, hb, hcache, stripA, stripB,
             copyA_sems, copyB_sems,
             txA, rxA, txB, rxB, txC, rxC, txD, rxD, txZ, rxZ,
             sendA, recvA, sendB, recvB,
             sendC, recvC, sendD, recvD, sendZ, recvZ,
             creditA, creditB, creditC, creditD, creditZ):
        my = lax.axis_index("i")
        left = lax.rem(my - 1 + N_DEV, N_DEV)
        right = lax.rem(my + 1, N_DEV)

        def cidx(off):
            return lax.rem(my + off + 4 * N_DEV, N_DEV)

        def chunk(idx):
            return pl.ds(idx * ch, ch)

        jobsA = [(w1_ref.at[:, pl.ds(j * STRIP, STRIP)], w1b, j * STRIP)
                 for j in range(h_dim // STRIP)]
        jobsA += [(x_ref.at[:, pl.ds(j * STRIP, STRIP)], xb, j * STRIP)
                  for j in range(k // STRIP)]
        jobsB = [(w2_ref.at[pl.ds(j * STRIP, STRIP), :], w2b, j * STRIP)
                 for j in range(h_dim // STRIP)]

        def strip_copy(jobs, buf, sems, j):
            return pltpu.make_async_copy(jobs[j][0], buf.at[j % 2],
                                         sems.at[j % 2])

        cpsA = [strip_copy(jobsA, stripA, copyA_sems, j) for j in range(2)]
        cpsB = [strip_copy(jobsB, stripB, copyB_sems, j) for j in range(2)]
        for cp in cpsA + cpsB:
            cp.start()

        barrier_sem = pltpu.get_barrier_semaphore()
        for nbr in (left, right, cidx(4)):
            pl.semaphore_signal(barrier_sem, inc=1, device_id=(nbr,),
                                device_id_type=pl.DeviceIdType.MESH)
        pl.semaphore_wait(barrier_sem, 3)

        for j in range(len(jobsA)):
            cpsA[j].wait()
            dst, off = jobsA[j][1], jobsA[j][2]
            dst[:, pl.ds(off, STRIP)] = stripA[j % 2].astype(jnp.bfloat16)
            if j + 2 < len(jobsA):
                cp = strip_copy(jobsA, stripA, copyA_sems, j + 2)
                cp.start()
                cpsA.append(cp)
        hb[:, :] = jnp.maximum(
            jnp.dot(xb[chunk(my), :], w1b[:, :],
                    preferred_element_type=jnp.float32), 0.0
        ).astype(jnp.bfloat16)
        for j in range(len(jobsB)):
            cpsB[j].wait()
            dst, off = jobsB[j][1], jobsB[j][2]
            dst[pl.ds(off, STRIP), :] = stripB[j % 2].astype(jnp.bfloat16)
            if j + 2 < len(jobsB):
                cp = strip_copy(jobsB, stripB, copyB_sems, j + 2)
                cp.start()
                cpsB.append(cp)

        def compute_chunk(c):
            rows = chunk(c)
            hc = jnp.dot(xb[rows, :], w1b[:, :],
                         preferred_element_type=jnp.float32)
            hc = jnp.maximum(hc, 0.0).astype(jnp.bfloat16)
            out_ref[rows, :] = jnp.dot(hc, w2b[:, :],
                                       preferred_element_type=jnp.float32)

        out_ref[chunk(my), :] = jnp.dot(hb[:, :], w2b[:, :],
                                        preferred_element_type=jnp.float32)

        def make_stream(tx, rx, ssems, rsems, credit, tgt, up, total):
            msgs = []
            ncons = [0]

            def send(src_chunk, col):
                j = len(msgs)
                if j >= 2:
                    msgs[j - 2].wait_send()
                    pl.semaphore_wait(credit, 1)
                tx[j % 2, :, :] = out_ref[chunk(src_chunk),
                                          pl.ds(col, nh)].astype(
                    jnp.bfloat16)
                r = pltpu.make_async_remote_copy(
                    src_ref=tx.at[j % 2], dst_ref=rx.at[j % 2],
                    send_sem=ssems.at[j % 2], recv_sem=rsems.at[j % 2],
                    device_id=(tgt,), device_id_type=pl.DeviceIdType.MESH)
                r.start()
                msgs.append(r)

            def consume(dst_chunk, col, accumulate):
                j = ncons[0]
                msgs[j].wait_recv()
                val = rx[j % 2, :, :].astype(jnp.float32)
                rows, cols = chunk(dst_chunk), pl.ds(col, nh)
                if accumulate:
                    out_ref[rows, cols] = out_ref[rows, cols] + val
                else:
                    out_ref[rows, cols] = val
                if j + 2 < total:
                    pl.semaphore_signal(credit, inc=1, device_id=(up,),
                                        device_id_type=pl.DeviceIdType.MESH)
                ncons[0] += 1

            def drain():
                msgs[-2].wait_send()
                msgs[-1].wait_send()

            return send, consume, drain

        LOW, HIGH = 0, nh
        sendA_, consA, drainA = make_stream(
            txA, rxA, sendA, recvA, creditA, right, left, 10)
        sendB_, consB, drainB = make_stream(
            txB, rxB, sendB, recvB, creditB, left, right, 10)
        sendC_, consC, drainC = make_stream(
            txC, rxC, sendC, recvC, creditC, left, right, 3)
        sendD_, consD, drainD = make_stream(
            txD, rxD, sendD, recvD, creditD, right, left, 3)
        sendZ_, consZ, drainZ = make_stream(
            txZ, rxZ, sendZ, recvZ, creditZ, cidx(4), cidx(4), 2)

        def h_of(c):
            return jnp.maximum(
                jnp.dot(xb[chunk(c), :], w1b[:, :],
                        preferred_element_type=jnp.float32), 0.0
            ).astype(jnp.bfloat16)

        def half_dot(hval, dst_chunk, col):
            out_ref[chunk(dst_chunk), pl.ds(col, nh)] = jnp.dot(
                hval, w2b[:, pl.ds(col, nh)],
                preferred_element_type=jnp.float32)

        for s in range(N_DEV - 1):
            sendA_(cidx(-s), LOW)
            sendB_(cidx(s), HIGH)
            if s < 3:
                hv = h_of(cidx(s + 1))
                hcache[s, :, :] = hv
                half_dot(hv, cidx(s + 1), HIGH)
                hv = h_of(cidx(-s - 1))
                hcache[3 + s, :, :] = hv
                half_dot(hv, cidx(-s - 1), LOW)
            elif s == 3:
                compute_chunk(cidx(4))
            else:
                q = 6 - s
                half_dot(hcache[q, :, :], cidx(q + 1), LOW)
                half_dot(hcache[3 + q, :, :], cidx(-q - 1), HIGH)
            consA(cidx(-s - 1), LOW, True)
            consB(cidx(s + 1), HIGH, True)


        sendZ_(cidx(1), LOW)
        sendZ_(cidx(-1), HIGH)
        for t in range(3):
            sendA_(cidx(1 - t), LOW)
            sendB_(cidx(t - 1), HIGH)
            sendC_(cidx(1 + t), LOW)
            sendD_(cidx(-1 - t), HIGH)
            consA(cidx(-t), LOW, False)
            consB(cidx(t), HIGH, False)
            consC(cidx(2 + t), LOW, False)
            consD(cidx(-2 - t), HIGH, False)
        consZ(cidx(5), LOW, False)
        consZ(cidx(3), HIGH, False)

        for drain in (drainA, drainB, drainC, drainD, drainZ):
            drain()

    return pl.pallas_call(
        body,
        out_shape=jax.ShapeDtypeStruct((m, n), jnp.float32),
        in_specs=[pl.BlockSpec(memory_space=pltpu.HBM)] * 3,
        out_specs=pl.BlockSpec(memory_space=pltpu.VMEM),
        scratch_shapes=[
            pltpu.VMEM((m, k), jnp.bfloat16),
            pltpu.VMEM((k, h_dim), jnp.bfloat16),
            pltpu.VMEM((h_dim, n), jnp.bfloat16),
            pltpu.VMEM((m // N_DEV, h_dim), jnp.bfloat16),
            pltpu.VMEM((6, m // N_DEV, h_dim), jnp.bfloat16),
            pltpu.VMEM((2, m, STRIP), jnp.float32),
            pltpu.VMEM((2, STRIP, n), jnp.float32),
            pltpu.SemaphoreType.DMA((2,)),
            pltpu.SemaphoreType.DMA((2,)),
            pltpu.VMEM((2, ch, nh), jnp.bfloat16),
            pltpu.VMEM((2, ch, nh), jnp.bfloat16),
            pltpu.VMEM((2, ch, nh), jnp.bfloat16),
            pltpu.VMEM((2, ch, nh), jnp.bfloat16),
            pltpu.VMEM((2, ch, nh), jnp.bfloat16),
            pltpu.VMEM((2, ch, nh), jnp.bfloat16),
            pltpu.VMEM((2, ch, nh), jnp.bfloat16),
            pltpu.VMEM((2, ch, nh), jnp.bfloat16),
            pltpu.VMEM((2, ch, nh), jnp.bfloat16),
            pltpu.VMEM((2, ch, nh), jnp.bfloat16),
            pltpu.SemaphoreType.DMA((2,)),
            pltpu.SemaphoreType.DMA((2,)),
            pltpu.SemaphoreType.DMA((2,)),
            pltpu.SemaphoreType.DMA((2,)),
            pltpu.SemaphoreType.DMA((2,)),
            pltpu.SemaphoreType.DMA((2,)),
            pltpu.SemaphoreType.DMA((2,)),
            pltpu.SemaphoreType.DMA((2,)),
            pltpu.SemaphoreType.DMA((2,)),
            pltpu.SemaphoreType.DMA((2,)),
            pltpu.SemaphoreType.REGULAR,
            pltpu.SemaphoreType.REGULAR,
            pltpu.SemaphoreType.REGULAR,
            pltpu.SemaphoreType.REGULAR,
            pltpu.SemaphoreType.REGULAR,
        ],
        compiler_params=pltpu.CompilerParams(
            collective_id=0, vmem_limit_bytes=100 * 1024 * 1024),
    )(x, W1, W2)
